# baseline (speedup 1.0000x reference)
import functools

import jax
import jax.numpy as jnp
from jax import lax
from jax.experimental import pallas as pl
from jax.experimental.pallas import tpu as pltpu
from jax.experimental.pallas import tpu_sc as plsc

GTF = 3
NCOMP = GTF * 3
L = 16
NC, NS = 2, 16
NW = NC * NS
CH = 128


def _make_sc_kernel(B, S, P):
    N = B * S
    assert N % NW == 0
    per_w = N // NW
    assert per_w % CH == 0
    nch = per_w // CH
    assert S % per_w == 0

    mesh = plsc.VectorSubcoreMesh(core_axis_name="c", subcore_axis_name="s")

    @functools.partial(
        pl.kernel,
        mesh=mesh,
        compiler_params=pltpu.CompilerParams(
            needs_layout_passes=False, use_tc_tiling_on_sc=False),
        out_type=[
            jax.ShapeDtypeStruct((NW, L), jnp.float32),
            jax.ShapeDtypeStruct((NW, L), jnp.float32),
        ],
        scratch_types=[
            pltpu.VMEM((nch, CH), jnp.int32),
            pltpu.VMEM((NCOMP, per_w), jnp.float32),
            pltpu.VMEM((per_w,), jnp.int32),
            pltpu.VMEM((3, per_w), jnp.float32),
            pltpu.VMEM((3, per_w), jnp.float32),
            pltpu.VMEM((L,), jnp.float32),
            pltpu.VMEM((L,), jnp.float32),
            pltpu.SemaphoreType.DMA,
            pltpu.SemaphoreType.DMA,
        ],
    )
    def sc_kernel(pidx_hbm, seed_hbm, vote_hbm, vl_hbm, mask_hbm,
                  num_hbm, den_hbm,
                  idx_v, gt_v, mask_v, seed_v, vote_v,
                  accn_v, accd_v, sem, xyz_sem):
        cid = lax.axis_index("c")
        sid = lax.axis_index("s")
        wid = sid * NC + cid
        base = wid * per_w
        b = wid // (S // per_w)
        s0 = (wid % (S // per_w)) * per_w

        for c in range(nch):
            pltpu.sync_copy(
                pidx_hbm.at[b, pl.ds(s0 + c * CH, CH)], idx_v.at[c])
        xyz_descs = []
        for k in range(3):
            xyz_descs.append(pltpu.async_copy(
                seed_hbm.at[k, pl.ds(base, per_w)], seed_v.at[k], xyz_sem))
            xyz_descs.append(pltpu.async_copy(
                vote_hbm.at[k, pl.ds(base, per_w)], vote_v.at[k], xyz_sem))

        descs = []
        for c in range(nch):
            dst = pl.ds(c * CH, CH)
            idx_c = idx_v.at[c]
            chunk_descs = [pltpu.async_copy(
                mask_hbm.at[b].at[idx_c], mask_v.at[dst], sem)]
            for k in range(NCOMP):
                chunk_descs.append(pltpu.async_copy(
                    vl_hbm.at[k, b].at[idx_c], gt_v.at[k, dst], sem))
            descs.append(chunk_descs)

        for dsc in xyz_descs:
            dsc.wait()

        zeros = jnp.zeros((L,), jnp.float32)

        def body(c, g, carry):
            num, den = carry
            sl = pl.ds(c * CH + g * L, L)
            sx = [seed_v[k, sl] for k in range(3)]
            vx = [vote_v[k, sl] for k in range(3)]
            d = None
            for j in range(GTF):
                dj = None
                for k in range(3):
                    t = jnp.abs(vx[k] - (gt_v[3 * j + k, sl] + sx[k]))
                    dj = t if dj is None else dj + t
                d = dj if d is None else jnp.minimum(d, dj)
            mf = mask_v[sl].astype(jnp.float32)
            return num + d * mf, den + mf

        acc = (zeros, zeros)
        for c in range(nch):
            for dsc in descs[c]:
                dsc.wait()
            acc = lax.fori_loop(
                0, CH // L, functools.partial(body, c), acc)
        num, den = acc

        accn_v[...] = num
        accd_v[...] = den
        pltpu.sync_copy(accn_v, num_hbm.at[wid])
        pltpu.sync_copy(accd_v, den_hbm.at[wid])

    return sc_kernel


def kernel(seed_xyz, vote_xyz, seed_inds, vote_label_mask, vote_label):
    B, S, _ = seed_xyz.shape
    P = vote_label.shape[1]
    N = B * S

    seed_t = jnp.transpose(seed_xyz, (2, 0, 1)).reshape(3, N)
    vote_t = jnp.transpose(vote_xyz, (2, 0, 1)).reshape(3, N)
    vl_t = jnp.transpose(vote_label, (2, 0, 1))
    pidx = seed_inds.astype(jnp.int32)
    mask2 = vote_label_mask.astype(jnp.int32)

    sc = _make_sc_kernel(B, S, P)
    num, den = sc(pidx, seed_t, vote_t, vl_t, mask2)
    return jnp.sum(num) / (jnp.sum(den) + 1e-6)
